# Initial kernel scaffold; baseline (speedup 1.0000x reference)
import jax
import jax.numpy as jnp
from jax.experimental import pallas as pl


def kernel(node_features, edge_index, norm1_w, norm1_b, norm2_w, norm2_b, msg_w1, msg_b1, msg_w2, msg_b2, upd_w1, upd_b1, upd_w2, upd_b2, gate_w, gate_b):
    raise NotImplementedError("write your pallas kernel here")


# speedup vs baseline: 3.1552x; 3.1552x over previous
import functools

import jax
import jax.numpy as jnp
from jax import lax
from jax.experimental import pallas as pl
from jax.experimental.pallas import tpu as pltpu
from jax.experimental.pallas import tpu_sc as plsc

_NC = 2
_NS = 16
_NT = _NC * _NS
_CR = 80
_CW = 16


@functools.lru_cache(maxsize=None)
def _gather_call(n, e, d):
    cpt = e // (_NT * _CR)
    mesh = plsc.VectorSubcoreMesh(core_axis_name="c", subcore_axis_name="s")

    @functools.partial(
        pl.kernel,
        mesh=mesh,
        out_type=[
            jax.ShapeDtypeStruct((e, d), jnp.float32),
            jax.ShapeDtypeStruct((e, d), jnp.float32),
        ],
        scratch_types=[
            pltpu.VMEM((cpt, _CR), jnp.int32),
            pltpu.VMEM((cpt, _CR), jnp.int32),
            pltpu.VMEM((_CR, d), jnp.float32),
            pltpu.VMEM((_CR, d), jnp.float32),
            pltpu.SemaphoreType.DMA,
            pltpu.SemaphoreType.DMA,
        ],
    )
    def gather_k(flat, src3d, tgt3d, sf, tf, sidx, tidx, srows, trows, sem1, sem2):
        c = lax.axis_index("c")
        s = lax.axis_index("s")
        wid = c * _NS + s
        pltpu.sync_copy(src3d.at[wid], sidx)
        pltpu.sync_copy(tgt3d.at[wid], tidx)

        def body(j, carry):
            base = (wid * cpt + j) * _CR
            cp1 = pltpu.async_copy(flat.at[sidx.at[j]], srows, sem1)
            cp2 = pltpu.async_copy(flat.at[tidx.at[j]], trows, sem2)
            cp1.wait()
            cp2.wait()
            pltpu.sync_copy(srows, sf.at[pl.ds(base, _CR)])
            pltpu.sync_copy(trows, tf.at[pl.ds(base, _CR)])
            return carry

        lax.fori_loop(0, cpt, body, 0)

    return gather_k


@functools.lru_cache(maxsize=None)
def _scatter_call(n, e, d):
    cpt = e // (_NT * _CR)
    grp = 5
    ngrp = cpt // grp
    nzc = n // _CR
    zc_max = -(-nzc // _NS)
    mesh = plsc.VectorSubcoreMesh(core_axis_name="c", subcore_axis_name="s")

    @functools.partial(
        pl.kernel,
        mesh=mesh,
        out_type=[
            jax.ShapeDtypeStruct((_NC * n, d), jnp.float32),
            jax.ShapeDtypeStruct((_NC * n, d), jnp.float32),
        ],
        scratch_types=[
            pltpu.VMEM_SHARED((n, d), jnp.float32),
            pltpu.VMEM((grp, _CR), jnp.int32),
            pltpu.VMEM((_CR, d), jnp.float32),
            pltpu.VMEM((_CR, d), jnp.float32),
        ],
    )
    def scatter_k(msgs, tgt4d, ones_h, z128_h, agg2, cnt2,
                  agg_sh, tidx, msg_v, ones_v):
        c = lax.axis_index("c")
        s = lax.axis_index("s")
        wid = c * _NS + s
        pltpu.sync_copy(ones_h, ones_v)

        def zero_acc():
            pltpu.sync_copy(z128_h, msg_v)
            for r in range(zc_max):
                k = s + _NS * r

                @pl.when(k < nzc)
                def _():
                    pltpu.sync_copy(msg_v, agg_sh.at[pl.ds(k * _CR, _CR)])

        def write_acc(dst):
            for r in range(zc_max):
                k = s + _NS * r

                @pl.when(k < nzc)
                def _():
                    pltpu.sync_copy(agg_sh.at[pl.ds(k * _CR, _CR)], msg_v)
                    pltpu.sync_copy(msg_v, dst.at[pl.ds(c * n + k * _CR, _CR)])

        zero_acc()
        plsc.subcore_barrier()

        def group(g, carry):
            pltpu.sync_copy(tgt4d.at[wid, g], tidx)
            for jj in range(grp):
                base = ((wid * ngrp + g) * grp + jj) * _CR
                pltpu.sync_copy(msgs.at[pl.ds(base, _CR)], msg_v)
                pltpu.sync_copy(msg_v, agg_sh.at[tidx.at[jj]], add=True)
            return carry

        lax.fori_loop(0, ngrp, group, 0)
        plsc.subcore_barrier()
        write_acc(agg2)

        zero_acc()
        plsc.subcore_barrier()

        def group2(g, carry):
            pltpu.sync_copy(tgt4d.at[wid, g], tidx)
            for jj in range(grp):
                pltpu.sync_copy(ones_v, agg_sh.at[tidx.at[jj]], add=True)
            return carry

        lax.fori_loop(0, ngrp, group2, 0)
        plsc.subcore_barrier()
        write_acc(cnt2)

    return scatter_k


def _gelu(x):
    return 0.5 * x * (1.0 + lax.erf(x * 0.7071067811865476))


def _ln(x, w, b, eps=1e-5):
    m = jnp.mean(x, axis=-1, keepdims=True)
    v = jnp.mean((x - m) ** 2, axis=-1, keepdims=True)
    return (x - m) * lax.rsqrt(v + eps) * w + b


def _mlp_body(sf_r, tf_r, w1a_r, w1b_r, b1_r, w2_r, b2_r, o_r):
    h = (jnp.dot(sf_r[...], w1a_r[...], preferred_element_type=jnp.float32)
         + jnp.dot(tf_r[...], w1b_r[...], preferred_element_type=jnp.float32)
         + b1_r[...])
    h = _gelu(h)
    o_r[...] = jnp.dot(h, w2_r[...], preferred_element_type=jnp.float32) + b2_r[...]


def _mlp_call(sf, tf, w1a, w1b, b1, w2, b2):
    e, d = sf.shape
    be = 512
    full2 = lambda shape: pl.BlockSpec(shape, lambda i: (0, 0))
    return pl.pallas_call(
        _mlp_body,
        grid=(e // be,),
        in_specs=[
            pl.BlockSpec((be, d), lambda i: (i, 0)),
            pl.BlockSpec((be, d), lambda i: (i, 0)),
            full2((d, 2 * d)),
            full2((d, 2 * d)),
            full2((1, 2 * d)),
            full2((2 * d, d)),
            full2((1, d)),
        ],
        out_specs=pl.BlockSpec((be, d), lambda i: (i, 0)),
        out_shape=jax.ShapeDtypeStruct((e, d), jnp.float32),
    )(sf, tf, w1a, w1b, b1, w2, b2)


def _node_body(nf_r, a0_r, a1_r, c0_r, c1_r, n1w_r, n1b_r, n2w_r, n2b_r,
               u1a_r, u1b_r, ub1_r, u2_r, ub2_r, gwa_r, gwb_r, gb_r, o_r):
    cnt = c0_r[...][:, 0:1] + c1_r[...][:, 0:1]
    agg = (a0_r[...] + a1_r[...]) / jnp.maximum(cnt, 1.0)
    x = nf_r[...]
    normed = _ln(x, n1w_r[...], n1b_r[...])
    h = (jnp.dot(normed, u1a_r[...], preferred_element_type=jnp.float32)
         + jnp.dot(agg, u1b_r[...], preferred_element_type=jnp.float32)
         + ub1_r[...])
    h = _gelu(h)
    upd = jnp.dot(h, u2_r[...], preferred_element_type=jnp.float32) + ub2_r[...]
    gate = jax.nn.sigmoid(
        jnp.dot(normed, gwa_r[...], preferred_element_type=jnp.float32)
        + jnp.dot(agg, gwb_r[...], preferred_element_type=jnp.float32)
        + gb_r[...])
    o_r[...] = _ln(x + gate * upd, n2w_r[...], n2b_r[...])


def _node_call(flat, a0, a1, c0, c1, n1w, n1b, n2w, n2b,
               u1a, u1b, ub1, u2, ub2, gwa, gwb, gb):
    n, d = flat.shape
    bn = 1000
    full2 = lambda shape: pl.BlockSpec(shape, lambda i: (0, 0))
    row = lambda w: pl.BlockSpec((bn, w), lambda i: (i, 0))
    return pl.pallas_call(
        _node_body,
        grid=(n // bn,),
        in_specs=[
            row(d), row(d), row(d), row(d), row(d),
            full2((1, d)), full2((1, d)), full2((1, d)), full2((1, d)),
            full2((d, 2 * d)), full2((d, 2 * d)), full2((1, 2 * d)),
            full2((2 * d, d)), full2((1, d)),
            full2((d, d)), full2((d, d)), full2((1, d)),
        ],
        out_specs=row(d),
        out_shape=jax.ShapeDtypeStruct((n, d), jnp.float32),
    )(flat, a0, a1, c0, c1, n1w, n1b, n2w, n2b,
      u1a, u1b, ub1, u2, ub2, gwa, gwb, gb)


def kernel(node_features, edge_index, norm1_w, norm1_b, norm2_w, norm2_b,
           msg_w1, msg_b1, msg_w2, msg_b2,
           upd_w1, upd_b1, upd_w2, upd_b2,
           gate_w, gate_b):
    b, n_per, d = node_features.shape
    flat = node_features.reshape(-1, d)
    n = flat.shape[0]
    offsets = (jnp.arange(b, dtype=edge_index.dtype) * n_per)[:, None]
    src = (edge_index[0][None, :] + offsets).reshape(-1)
    tgt = (edge_index[1][None, :] + offsets).reshape(-1)
    e = src.shape[0]

    cpt = e // (_NT * _CR)
    src3d = src.reshape(_NT, cpt, _CR)
    tgt3d = tgt.reshape(_NT, cpt, _CR)

    sf, tf = _gather_call(n, e, d)(flat, src3d, tgt3d)

    w1a = msg_w1[:, :d].T
    w1b = msg_w1[:, d:].T
    w2m = msg_w2.T
    msgs = _mlp_call(sf, tf, w1a, w1b, msg_b1.reshape(1, -1), w2m,
                     msg_b2.reshape(1, -1))

    ones_h = jnp.ones((_CR, d), jnp.float32)
    z128_h = jnp.zeros((_CR, d), jnp.float32)
    tgt4d = tgt.reshape(_NT, cpt // 5, 5, _CR)
    aggs, cnts = _scatter_call(n, e, d)(msgs, tgt4d, ones_h, z128_h)

    out = _node_call(
        flat, aggs[:n], aggs[n:], cnts[:n], cnts[n:],
        norm1_w.reshape(1, -1), norm1_b.reshape(1, -1),
        norm2_w.reshape(1, -1), norm2_b.reshape(1, -1),
        upd_w1[:, :d].T, upd_w1[:, d:].T, upd_b1.reshape(1, -1),
        upd_w2.T, upd_b2.reshape(1, -1),
        gate_w[:, :d].T, gate_w[:, d:].T, gate_b.reshape(1, -1))
    return out.reshape(b, n_per, d)
